# Initial kernel scaffold; baseline (speedup 1.0000x reference)
import jax
import jax.numpy as jnp
from jax.experimental import pallas as pl


def kernel(x, edge_index, W1, b1, W2, b2):
    raise NotImplementedError("write your pallas kernel here")


# speedup vs baseline: 15.7588x; 15.7588x over previous
import jax
import jax.numpy as jnp
from jax import lax
from jax.experimental import pallas as pl
from jax.experimental.pallas import tpu as pltpu
from jax.experimental.pallas import tpu_sc as plsc

K_HOPS = 10
ALPHA = 0.1


def _mlp_body(x_ref, w1_ref, b1_ref, w2_ref, b2_ref, out_ref):
    h = jnp.dot(x_ref[...], w1_ref[...], preferred_element_type=jnp.float32)
    h = jnp.maximum(h + b1_ref[...], 0.0)
    out_ref[...] = (
        jnp.dot(h, w2_ref[...], preferred_element_type=jnp.float32) + b2_ref[...]
    )


def _logsoftmax_body(g0_ref, g1_ref, sq_ref, out_ref):
    f = jnp.concatenate([g0_ref[0], g1_ref[0]], axis=-1) * sq_ref[...]
    m = jnp.max(f, axis=-1, keepdims=True)
    e = jnp.exp(f - m)
    out_ref[...] = (f - m) - jnp.log(jnp.sum(e, axis=-1, keepdims=True))


def _make_sc_kernel(NP, NT, CH, HALF):
    HP = NP // 2
    HR = HP // 128
    NR = NT // 128
    CG = CH // 8
    OMA = 1.0 - ALPHA
    mesh = plsc.VectorSubcoreMesh(core_axis_name="c", subcore_axis_name="s")

    def body(h_ref, src_ref, dst_ref, g_out, sq_out,
             sidx, didx, g_sl, h2_sl, aggb, rowbuf,
             hist, zbuf, idxdeg, dout2d, din2d, sq2d, w1d,
             g_sh, agg_sh, deg_sh):
        c = lax.axis_index("c")
        s = lax.axis_index("s")
        r0 = s * NT

        zeros16 = jnp.zeros((16,), jnp.float32)
        ones16 = jnp.ones((16,), jnp.float32)
        iota16 = lax.iota(jnp.int32, 16)

        def vloop(n, f):
            lax.fori_loop(0, n, lambda i, _: (f(i), 0)[1], 0)

        def zz(i):
            zbuf[i >> 3, pl.ds((i & 7) * 16, 16)] = zeros16
        vloop(128, zz)
        pltpu.sync_copy(zbuf, deg_sh.at[pl.ds(16 * s, 16)])
        pltpu.sync_copy(h_ref.at[c, pl.ds(r0, NT)], aggb)
        plsc.subcore_barrier()

        def hscan(idx_ref, lo, base_row):
            def zh(i):
                hist[i >> 3, pl.ds((i & 7) * 16, 16)] = zeros16
            vloop((HR + 8) * 8, zh)

            def grp(gi):
                pltpu.sync_copy(idx_ref.at[pl.ds(s * CH + gi * 8, 8)], sidx)

                def vstep(i):
                    v = sidx[i >> 3, pl.ds((i & 7) * 16, 16)] - lo
                    m = (v >= 0) & (v < HP)
                    plsc.addupdate_scatter(
                        hist, [v >> 7, v & 127], ones16, mask=m)
                vloop(64, vstep)
            vloop(CG, grp)

            def widx(i):
                idxdeg[0, pl.ds(i * 16, 16)] = iota16 + (base_row + i * 16)
            vloop((HR + 8) // 16, widx)
            pltpu.sync_copy(hist, deg_sh.at[idxdeg.at[0]], add=True)

        hscan(src_ref, 0, 0)
        hscan(src_ref, HP, HR)
        hscan(dst_ref, 0, 128)
        hscan(dst_ref, HP, 128 + HR)
        plsc.subcore_barrier()

        pltpu.sync_copy(deg_sh.at[pl.ds(NR * s, NR)], dout2d)
        pltpu.sync_copy(deg_sh.at[pl.ds(128 + NR * s, NR)], din2d)

        def rsqrt16(d):
            i = plsc.bitcast(d, jnp.int32)
            y = plsc.bitcast(jnp.int32(0x5F3759DF) - (i >> 1), jnp.float32)
            for _ in range(3):
                y = y * (1.5 - 0.5 * d * y * y)
            return y

        def nv(j):
            r = j >> 3
            sl = pl.ds((j & 7) * 16, 16)
            d = dout2d[r, sl] + 1.0
            y = rsqrt16(d)
            sq2d[r, sl] = d * y
            dout2d[r, sl] = y
            din2d[r, sl] = rsqrt16(din2d[r, sl] + 1.0)
            w1d[pl.ds(j * 16, 16)] = y * din2d[r, sl]
        vloop(NT // 16, nv)

        @pl.when(c == 0)
        def _():
            pltpu.sync_copy(sq2d, sq_out.at[pl.ds(NR * s, NR)])

        def irow(i):
            rv = jnp.full((16,), i >> 7, jnp.int32)
            cv = jnp.full((16,), i & 127, jnp.int32)
            dsp = plsc.load_gather(dout2d, [rv, cv])
            for off in (0, HALF // 2):
                sl = pl.ds(off, HALF // 2)
                gv = dsp * aggb[i, sl]
                g_sl[i, sl] = gv
                h2_sl[i, sl] = ALPHA * gv
        vloop(NT, irow)

        def zrow(i):
            aggb[i, pl.ds(0, HALF // 2)] = zeros16
            aggb[i, pl.ds(HALF // 2, HALF // 2)] = zeros16
        vloop(NT, zrow)
        pltpu.sync_copy(aggb, agg_sh.at[pl.ds(r0, NT)])
        pltpu.sync_copy(g_sl, g_sh.at[pl.ds(r0, NT)])
        plsc.subcore_barrier()

        def kstep(k, _):
            def grp(gi):
                base = s * CH + gi * 8
                pltpu.sync_copy(src_ref.at[pl.ds(base, 8)], sidx)
                pltpu.sync_copy(dst_ref.at[pl.ds(base, 8)], didx)
                for jj in range(8):
                    pltpu.sync_copy(g_sh.at[sidx.at[jj]], rowbuf)
                    pltpu.sync_copy(rowbuf, agg_sh.at[didx.at[jj]], add=True)
            vloop(CG, grp)
            plsc.subcore_barrier()

            pltpu.sync_copy(agg_sh.at[pl.ds(r0, NT)], aggb)

            def crow(i):
                wv = plsc.load_gather(w1d, [jnp.full((16,), i, jnp.int32)])
                for off in (0, HALF // 2):
                    sl = pl.ds(off, HALF // 2)
                    g_sl[i, sl] = (
                        OMA * wv * (aggb[i, sl] + g_sl[i, sl]) + h2_sl[i, sl]
                    )
                    aggb[i, sl] = zeros16
            vloop(NT, crow)

            pltpu.sync_copy(aggb, agg_sh.at[pl.ds(r0, NT)])
            pltpu.sync_copy(g_sl, g_sh.at[pl.ds(r0, NT)])
            plsc.subcore_barrier()
            return 0
        lax.fori_loop(0, K_HOPS, kstep, 0)

        pltpu.sync_copy(g_sl, g_out.at[c, pl.ds(r0, NT)])

    return pl.kernel(
        body,
        out_type=(
            jax.ShapeDtypeStruct((2, NP, HALF), jnp.float32),
            jax.ShapeDtypeStruct((NP // 128, 128), jnp.float32),
        ),
        mesh=mesh,
        compiler_params=pltpu.CompilerParams(
            needs_layout_passes=False, use_tc_tiling_on_sc=False),
        scratch_types=[
            pltpu.VMEM((8, 128), jnp.int32),
            pltpu.VMEM((8, 128), jnp.int32),
            pltpu.VMEM((NT, HALF), jnp.float32),
            pltpu.VMEM((NT, HALF), jnp.float32),
            pltpu.VMEM((NT, HALF), jnp.float32),
            pltpu.VMEM((128, HALF), jnp.float32),
            pltpu.VMEM((HR + 8, 128), jnp.float32),
            pltpu.VMEM((16, 128), jnp.float32),
            pltpu.VMEM((1, HR + 8), jnp.int32),
            pltpu.VMEM((NR, 128), jnp.float32),
            pltpu.VMEM((NR, 128), jnp.float32),
            pltpu.VMEM((NR, 128), jnp.float32),
            pltpu.VMEM((NT,), jnp.float32),
            pltpu.VMEM_SHARED((NP, HALF), jnp.float32),
            pltpu.VMEM_SHARED((NP, HALF), jnp.float32),
            pltpu.VMEM_SHARED((256, 128), jnp.float32),
        ],
    )


def kernel(x, edge_index, W1, b1, W2, b2):
    N, D = x.shape
    H = W1.shape[1]
    C = W2.shape[1]
    E = edge_index.shape[1]
    HALF = C // 2

    NT = -(-(N + 1) // (16 * 128)) * 128
    NP = 16 * NT
    CH = -(-(-(-E // 16)) // 1024) * 8
    EP = 16 * CH * 128

    BR = NT
    x_pad = jnp.concatenate([x, jnp.zeros((NP - N, D), x.dtype)])
    h = pl.pallas_call(
        _mlp_body,
        grid=(NP // BR,),
        in_specs=[
            pl.BlockSpec((BR, D), lambda i: (i, 0)),
            pl.BlockSpec((D, H), lambda i: (0, 0)),
            pl.BlockSpec((1, H), lambda i: (0, 0)),
            pl.BlockSpec((H, C), lambda i: (0, 0)),
            pl.BlockSpec((1, C), lambda i: (0, 0)),
        ],
        out_specs=pl.BlockSpec((BR, C), lambda i: (i, 0)),
        out_shape=jax.ShapeDtypeStruct((NP, C), jnp.float32),
    )(x_pad, W1, b1.reshape(1, H), W2, b2.reshape(1, C))
    h_pair = jnp.stack([h[:, :HALF], h[:, HALF:]])

    pad = jnp.full((EP - E,), N, jnp.int32)
    src2d = jnp.concatenate([edge_index[0], pad]).reshape(-1, 128)
    dst2d = jnp.concatenate([edge_index[1], pad]).reshape(-1, 128)

    g_pair, sq = _make_sc_kernel(NP, NT, CH, HALF)(h_pair, src2d, dst2d)

    out_pad = pl.pallas_call(
        _logsoftmax_body,
        grid=(NP // BR,),
        in_specs=[
            pl.BlockSpec((1, BR, HALF), lambda i: (0, i, 0)),
            pl.BlockSpec((1, BR, HALF), lambda i: (1, i, 0)),
            pl.BlockSpec((BR, 1), lambda i: (i, 0)),
        ],
        out_specs=pl.BlockSpec((BR, C), lambda i: (i, 0)),
        out_shape=jax.ShapeDtypeStruct((NP, C), jnp.float32),
    )(g_pair, g_pair, sq.reshape(NP, 1))
    return out_pad[:N]
